# baseline (speedup 1.0000x reference)
import functools

import jax
import jax.numpy as jnp
from jax import lax
from jax.experimental import pallas as pl
from jax.experimental.pallas import tpu as pltpu
from jax.experimental.pallas import tpu_sc as plsc

_S, _B, _D = 8192, 2, 1024
_NC, _NS = 2, 16
_NW = _NC * _NS
_PW = _S // _NW
_C = 8
_NCH = _PW // _C
_NBUF = 3
_NFULL = (_NCH // _NBUF) * _NBUF
_LANES = 16


def _sc_body(emb_hbm, pe_hbm, out_hbm, emb_bufs, pe_bufs, out_bufs, sins, souts):
    wid = lax.axis_index("s") * _NC + lax.axis_index("c")
    base = wid * _PW

    def start_in(g, b):
        s0 = base + g * _C
        pltpu.async_copy(emb_hbm.at[pl.ds(s0, _C)], emb_bufs[b], sins[b])
        pltpu.async_copy(pe_hbm.at[pl.ds(s0, _C)], pe_bufs[b], sins[b])

    def wait_in(b):
        pltpu.make_async_copy(emb_hbm.at[pl.ds(base, _C)], emb_bufs[b], sins[b]).wait()
        pltpu.make_async_copy(pe_hbm.at[pl.ds(base, _C)], pe_bufs[b], sins[b]).wait()

    def start_out(g, b):
        s0 = base + g * _C
        pltpu.async_copy(out_bufs[b], out_hbm.at[pl.ds(s0, _C)], souts[b])

    def wait_out(b):
        pltpu.make_async_copy(out_bufs[b], out_hbm.at[pl.ds(base, _C)], souts[b]).wait()

    def process(g, b):
        g = jnp.int32(g)
        wait_in(b)

        @pl.when(g >= _NBUF)
        def _():
            wait_out(b)

        @plsc.parallel_loop(0, _D // _LANES, unroll=8)
        def lane(j):
            off = j * _LANES
            for i in range(_C):
                pe_vec = pe_bufs[b][i, pl.ds(off, _LANES)]
                out_bufs[b][i, 0, pl.ds(off, _LANES)] = (
                    emb_bufs[b][i, 0, pl.ds(off, _LANES)] + pe_vec)
                out_bufs[b][i, 1, pl.ds(off, _LANES)] = (
                    emb_bufs[b][i, 1, pl.ds(off, _LANES)] + pe_vec)

        start_out(g, b)

        @pl.when(g + _NBUF < _NCH)
        def _():
            start_in(g + _NBUF, b)

    for b in range(_NBUF):
        start_in(b, b)

    def outer(k, _):
        for b in range(_NBUF):
            process(_NBUF * k + b, b)
        return 0

    lax.fori_loop(0, _NFULL // _NBUF, outer, 0)
    for g in range(_NFULL, _NCH):
        process(g, g % _NBUF)
    for b in range(_NBUF):
        wait_out(b)


def kernel(emb, pe_table):
    sc_kernel = pl.kernel(
        _sc_body,
        out_type=jax.ShapeDtypeStruct((_S, _B, _D), jnp.float32),
        mesh=plsc.VectorSubcoreMesh(core_axis_name="c", subcore_axis_name="s"),
        scratch_types=[
            [pltpu.VMEM((_C, _B, _D), jnp.float32) for _ in range(_NBUF)],
            [pltpu.VMEM((_C, _D), jnp.float32) for _ in range(_NBUF)],
            [pltpu.VMEM((_C, _B, _D), jnp.float32) for _ in range(_NBUF)],
            [pltpu.SemaphoreType.DMA for _ in range(_NBUF)],
            [pltpu.SemaphoreType.DMA for _ in range(_NBUF)],
        ],
    )
    return sc_kernel(emb, pe_table)


# speedup vs baseline: 1.0299x; 1.0299x over previous
import functools

import jax
import jax.numpy as jnp
from jax import lax
from jax.experimental import pallas as pl
from jax.experimental.pallas import tpu as pltpu
from jax.experimental.pallas import tpu_sc as plsc

_S, _B, _D = 8192, 2, 1024
_NC, _NS = 2, 16
_NW = _NC * _NS
_PW = _S // _NW
_C = 4
_NCH = _PW // _C
_NBUF = 6
_NFULL = (_NCH // _NBUF) * _NBUF
_LANES = 16


def _sc_body(emb_hbm, pe_hbm, out_hbm, emb_bufs, pe_bufs, out_bufs, sins, souts):
    wid = lax.axis_index("s") * _NC + lax.axis_index("c")
    base = wid * _PW

    def start_in(g, b):
        s0 = base + g * _C
        pltpu.async_copy(emb_hbm.at[pl.ds(s0, _C)], emb_bufs[b], sins[b])
        pltpu.async_copy(pe_hbm.at[pl.ds(s0, _C)], pe_bufs[b], sins[b])

    def wait_in(b):
        pltpu.make_async_copy(emb_hbm.at[pl.ds(base, _C)], emb_bufs[b], sins[b]).wait()
        pltpu.make_async_copy(pe_hbm.at[pl.ds(base, _C)], pe_bufs[b], sins[b]).wait()

    def start_out(g, b):
        s0 = base + g * _C
        pltpu.async_copy(out_bufs[b], out_hbm.at[pl.ds(s0, _C)], souts[b])

    def wait_out(b):
        pltpu.make_async_copy(out_bufs[b], out_hbm.at[pl.ds(base, _C)], souts[b]).wait()

    def process(g, b):
        g = jnp.int32(g)
        wait_in(b)

        @pl.when(g >= _NBUF)
        def _():
            wait_out(b)

        @plsc.parallel_loop(0, _D // _LANES, unroll=8)
        def lane(j):
            off = j * _LANES
            for i in range(_C):
                pe_vec = pe_bufs[b][i, pl.ds(off, _LANES)]
                out_bufs[b][i, 0, pl.ds(off, _LANES)] = (
                    emb_bufs[b][i, 0, pl.ds(off, _LANES)] + pe_vec)
                out_bufs[b][i, 1, pl.ds(off, _LANES)] = (
                    emb_bufs[b][i, 1, pl.ds(off, _LANES)] + pe_vec)

        start_out(g, b)

        @pl.when(g + _NBUF < _NCH)
        def _():
            start_in(g + _NBUF, b)

    for b in range(_NBUF):
        start_in(b, b)

    def outer(k, _):
        for b in range(_NBUF):
            process(_NBUF * k + b, b)
        return 0

    lax.fori_loop(0, _NFULL // _NBUF, outer, 0)
    for g in range(_NFULL, _NCH):
        process(g, g % _NBUF)
    for b in range(_NBUF):
        wait_out(b)


def kernel(emb, pe_table):
    sc_kernel = pl.kernel(
        _sc_body,
        out_type=jax.ShapeDtypeStruct((_S, _B, _D), jnp.float32),
        mesh=plsc.VectorSubcoreMesh(core_axis_name="c", subcore_axis_name="s"),
        scratch_types=[
            [pltpu.VMEM((_C, _B, _D), jnp.float32) for _ in range(_NBUF)],
            [pltpu.VMEM((_C, _D), jnp.float32) for _ in range(_NBUF)],
            [pltpu.VMEM((_C, _B, _D), jnp.float32) for _ in range(_NBUF)],
            [pltpu.SemaphoreType.DMA for _ in range(_NBUF)],
            [pltpu.SemaphoreType.DMA for _ in range(_NBUF)],
        ],
    )
    return sc_kernel(emb, pe_table)
